# baseline (speedup 1.0000x reference)
import jax
import jax.numpy as jnp
from jax import lax
from jax.experimental import pallas as pl
from jax.experimental.pallas import tpu as pltpu
from jax.experimental.pallas import tpu_sc as plsc

BATCH = 4096
SEQ = 200
HID = 64
ROWS = BATCH * SEQ
NC = 2
NS = 16
NW = NC * NS
LANES = 16
KREG = HID // LANES

VTAB = 1000000
CTILES = (VTAB + 127) // 128
TLINES = CTILES * 128

CHUNK = 256
NCHUNK = ROWS // (NW * CHUNK)
GPC = CHUNK // 128
IDXROWS = NCHUNK * GPC


BLKC = 2
NBLK = CTILES // BLKC


def _pack_body(tab_t_hbm, tlin_hbm, blk_v, line_v,
               isem0, isem1, osem0, osem1):
    c = lax.axis_index("c")
    s = lax.axis_index("s")
    wid = s * NC + c
    nblk = (NBLK - 1 - wid) // NW + 1

    isems = (isem0, isem1)
    osems = (osem0, osem1)
    iotas = [lax.iota(jnp.int32, LANES) + k * LANES for k in range(KREG)]
    W = BLKC * 128

    def istart(i, slot, width):
        col0 = pl.multiple_of((i * NW + wid) * W, 128)
        for ht in range(8):
            pltpu.async_copy(
                tab_t_hbm.at[pl.ds(ht * 8, 8), pl.ds(col0, width)],
                blk_v.at[slot, pl.ds(ht * 8, 8), pl.ds(0, width)],
                isems[slot])

    def iwait(slot, width):
        for ht in range(8):
            pltpu.make_async_copy(
                tab_t_hbm.at[pl.ds(ht * 8, 8), pl.ds(0, width)],
                blk_v.at[slot, pl.ds(ht * 8, 8), pl.ds(0, width)],
                isems[slot]).wait()

    def transpose(slot, width):
        def line(j, carry):
            cidx = jnp.full((LANES,), j, jnp.int32)
            for k in range(KREG):
                v = plsc.load_gather(blk_v.at[slot], [iotas[k], cidx])
                line_v[slot, j, pl.ds(k * LANES, LANES)] = v
            return carry
        lax.fori_loop(0, width, line, 0, unroll=8)

    def ostart(i, slot, width):
        base = (i * NW + wid) * W
        pltpu.async_copy(line_v.at[slot, pl.ds(0, width), :],
                         tlin_hbm.at[pl.ds(base, width)], osems[slot])

    def owait(slot, width):
        pltpu.make_async_copy(line_v.at[slot, pl.ds(0, width), :],
                              tlin_hbm.at[pl.ds(0, width)],
                              osems[slot]).wait()

    istart(0, 0, W)

    def step(i, carry):
        def one(slot):
            @pl.when(i + 1 < nblk)
            def _():
                istart(i + 1, 1 - slot, W)
            iwait(slot, W)

            @pl.when(i >= 2)
            def _():
                owait(slot, W)
            transpose(slot, W)
            ostart(i, slot, W)

        @pl.when(lax.rem(i, 2) == 0)
        def _():
            one(0)

        @pl.when(lax.rem(i, 2) == 1)
        def _():
            one(1)
        return carry

    lax.fori_loop(0, nblk, step, 0)
    owait(0, W)
    owait(1, W)

    @pl.when(wid == 0)
    def _():
        col0 = pl.multiple_of(NBLK * BLKC * 128, 128)
        for ht in range(8):
            pltpu.async_copy(
                tab_t_hbm.at[pl.ds(ht * 8, 8), pl.ds(col0, 128)],
                blk_v.at[0, pl.ds(ht * 8, 8), pl.ds(0, 128)],
                isems[0])
        iwait(0, 128)
        transpose(0, 128)
        pltpu.async_copy(line_v.at[0, pl.ds(0, 128), :],
                         tlin_hbm.at[pl.ds(NBLK * BLKC * 128, 128)],
                         osems[0])
        owait(0, 128)


def _emb_body(ids_hbm, tlin_hbm, pos_hbm, out_hbm,
              idx_v, rows_v, pos_v, gsem0, gsem1, osem0, osem1):
    c = lax.axis_index("c")
    s = lax.axis_index("s")
    wid = s * NC + c
    pltpu.sync_copy(pos_hbm, pos_v)
    pltpu.sync_copy(ids_hbm.at[pl.ds(wid * IDXROWS, IDXROWS)], idx_v)

    gsems = (gsem0, gsem1)
    osems = (osem0, osem1)

    def fire(j, slot):
        for g in range(GPC):
            pltpu.async_copy(
                tlin_hbm.at[idx_v.at[j * GPC + g]],
                rows_v.at[slot, pl.ds(g * 128, 128), :],
                gsems[slot])

    def gwait(slot):
        for g in range(GPC):
            pltpu.make_async_copy(
                tlin_hbm.at[idx_v.at[0]],
                rows_v.at[slot, pl.ds(g * 128, 128), :],
                gsems[slot]).wait()

    def compute(j, slot):
        p0 = lax.rem(j * CHUNK, SEQ)

        def rbody(i, p):
            pline = p // 2
            poff = lax.rem(p, 2) * HID
            for k in range(KREG):
                b = pos_v[pline, pl.ds(poff + k * LANES, LANES)]
                plsc.addupdate(rows_v.at[slot, i, pl.ds(k * LANES, LANES)], b)
            pn = p + 1
            return jnp.where(pn == SEQ, 0, pn)

        lax.fori_loop(0, CHUNK, rbody, p0, unroll=4)

    def ostart(j, slot):
        base = (wid * NCHUNK + j) * CHUNK
        pltpu.async_copy(rows_v.at[slot], out_hbm.at[pl.ds(base, CHUNK)],
                         osems[slot])

    def owait(slot):
        pltpu.make_async_copy(rows_v.at[slot], out_hbm.at[pl.ds(0, CHUNK)],
                              osems[slot]).wait()

    fire(0, 0)

    def step(j, carry):
        def one(slot):
            @pl.when(j + 1 < NCHUNK)
            def _():
                @pl.when(j >= 1)
                def _():
                    owait(1 - slot)
                fire(j + 1, 1 - slot)
            gwait(slot)
            compute(j, slot)
            ostart(j, slot)

        @pl.when(lax.rem(j, 2) == 0)
        def _():
            one(0)

        @pl.when(lax.rem(j, 2) == 1)
        def _():
            one(1)
        return carry

    lax.fori_loop(0, NCHUNK, step, 0)
    owait(0)
    owait(1)


def kernel(input_ids, item_table, pos_table):
    mesh = plsc.VectorSubcoreMesh(core_axis_name="c", subcore_axis_name="s")
    cp = pltpu.CompilerParams(use_tc_tiling_on_sc=True,
                              needs_layout_passes=False)

    pack = pl.kernel(
        _pack_body,
        out_type=jax.ShapeDtypeStruct((TLINES, 128), jnp.float32),
        mesh=mesh,
        compiler_params=cp,
        scratch_types=[
            pltpu.VMEM((2, HID, BLKC * 128), jnp.float32),
            pltpu.VMEM((2, BLKC * 128, 128), jnp.float32),
            pltpu.SemaphoreType.DMA,
            pltpu.SemaphoreType.DMA,
            pltpu.SemaphoreType.DMA,
            pltpu.SemaphoreType.DMA,
        ],
    )
    tlin = pack(item_table.T)

    ids2 = input_ids.reshape(ROWS // 128, 128).astype(jnp.int32)
    pos2 = pos_table.reshape(SEQ // 2, 128)

    emb = pl.kernel(
        _emb_body,
        out_type=jax.ShapeDtypeStruct((ROWS, 128), jnp.float32),
        mesh=mesh,
        compiler_params=cp,
        scratch_types=[
            pltpu.VMEM((IDXROWS, 128), jnp.int32),
            pltpu.VMEM((2, CHUNK, 128), jnp.float32),
            pltpu.VMEM((SEQ // 2, 128), jnp.float32),
            pltpu.SemaphoreType.DMA,
            pltpu.SemaphoreType.DMA,
            pltpu.SemaphoreType.DMA,
            pltpu.SemaphoreType.DMA,
        ],
    )
    out = emb(ids2, tlin, pos2)
    return out[:, :HID].reshape(BATCH, SEQ, HID)


# speedup vs baseline: 1.4497x; 1.4497x over previous
import jax
import jax.numpy as jnp
from jax import lax
from jax.experimental import pallas as pl
from jax.experimental.pallas import tpu as pltpu
from jax.experimental.pallas import tpu_sc as plsc

BATCH = 4096
SEQ = 200
HID = 64
ROWS = BATCH * SEQ
NC = 2
NS = 16
NW = NC * NS
CHUNK = 256
NCHUNK = ROWS // (NW * CHUNK)
GPC = CHUNK // 128
LANES = 16
VPR = HID // LANES


def _emb_body(ids_hbm, table_hbm, pos_hbm, out_hbm,
              idx_v, g_v, o_v, pos_v, gsem0, gsem1, osem0, osem1):
    c = lax.axis_index("c")
    s = lax.axis_index("s")
    wid = s * NC + c
    pltpu.sync_copy(pos_hbm, pos_v)

    gsems = (gsem0, gsem1)
    osems = (osem0, osem1)

    def fire(j, slot):
        pltpu.sync_copy(ids_hbm.at[wid * NCHUNK + j], idx_v.at[slot])
        for g in range(GPC):
            pltpu.async_copy(
                table_hbm.at[idx_v.at[slot, g]],
                g_v.at[slot, pl.ds(g * 128, 128), :],
                gsems[slot])

    def gwait(slot):
        for g in range(GPC):
            pltpu.make_async_copy(
                table_hbm.at[idx_v.at[slot, g]],
                g_v.at[slot, pl.ds(g * 128, 128), :],
                gsems[slot]).wait()

    def compute(j, slot):
        p0 = lax.rem(j * CHUNK, SEQ)

        def rbody(i, p):
            for v in range(VPR):
                a = g_v[slot, i, pl.ds(v * LANES, LANES)]
                b = pos_v[p, pl.ds(v * LANES, LANES)]
                o_v[slot, i, pl.ds(v * LANES, LANES)] = a + b
            pn = p + 1
            return jnp.where(pn == SEQ, 0, pn)

        lax.fori_loop(0, CHUNK, rbody, p0)

    def ostart(j, slot):
        base = (wid * NCHUNK + j) * CHUNK
        pltpu.async_copy(o_v.at[slot], out_hbm.at[pl.ds(base, CHUNK)],
                         osems[slot])

    def owait(slot):
        pltpu.make_async_copy(o_v.at[slot], out_hbm.at[pl.ds(0, CHUNK)],
                              osems[slot]).wait()

    fire(0, 0)

    def step(i, carry):
        j0 = 2 * i

        @pl.when(j0 > 0)
        def _():
            owait(1)
        fire(j0 + 1, 1)
        gwait(0)
        compute(j0, 0)
        ostart(j0, 0)

        @pl.when(j0 + 2 < NCHUNK)
        def _():
            owait(0)
            fire(j0 + 2, 0)
        gwait(1)
        compute(j0 + 1, 1)
        ostart(j0 + 1, 1)
        return carry

    lax.fori_loop(0, NCHUNK // 2, step, 0)
    owait(0)
    owait(1)


def kernel(input_ids, item_table, pos_table):
    ids = input_ids.reshape(NW * NCHUNK, GPC, 128).astype(jnp.int32)
    mesh = plsc.VectorSubcoreMesh(core_axis_name="c", subcore_axis_name="s")
    f = pl.kernel(
        _emb_body,
        out_type=jax.ShapeDtypeStruct((ROWS, 128), jnp.float32),
        mesh=mesh,
        compiler_params=pltpu.CompilerParams(use_tc_tiling_on_sc=False),
        scratch_types=[
            pltpu.VMEM((2, GPC, 128), jnp.int32),
            pltpu.VMEM((2, CHUNK, HID), jnp.float32),
            pltpu.VMEM((2, CHUNK, 128), jnp.float32),
            pltpu.VMEM((SEQ, HID), jnp.float32),
            pltpu.SemaphoreType.DMA,
            pltpu.SemaphoreType.DMA,
            pltpu.SemaphoreType.DMA,
            pltpu.SemaphoreType.DMA,
        ],
    )
    out = f(ids, item_table, pos_table)
    return out[:, :HID].reshape(BATCH, SEQ, HID)


# speedup vs baseline: 1.7420x; 1.2017x over previous
import jax
import jax.numpy as jnp
from jax import lax
from jax.experimental import pallas as pl
from jax.experimental.pallas import tpu as pltpu
from jax.experimental.pallas import tpu_sc as plsc

BATCH = 4096
SEQ = 200
HID = 64
ROWS = BATCH * SEQ
NC = 2
NS = 16
NW = NC * NS
CHUNK = 800
NCHUNK = ROWS // (NW * CHUNK)
NGATHER = 8
GLEN = CHUNK // NGATHER
SEQ_PER_CHUNK = CHUNK // SEQ
LANES = 16
VPR = HID // LANES


def _emb_body(ids_hbm, table_hbm, pos_hbm, out_hbm,
              idx_v, rows_v, pos_v, gsem0, gsem1, osem0, osem1):
    c = lax.axis_index("c")
    s = lax.axis_index("s")
    wid = s * NC + c
    pltpu.sync_copy(pos_hbm, pos_v)

    gsems = (gsem0, gsem1)
    osems = (osem0, osem1)

    def fire(j, slot):
        pltpu.sync_copy(ids_hbm.at[wid * NCHUNK + j], idx_v.at[slot])
        for g in range(NGATHER):
            pltpu.async_copy(
                table_hbm.at[idx_v.at[slot, g]],
                rows_v.at[slot, pl.ds(g * GLEN, GLEN), :],
                gsems[slot])

    def gwait(slot):
        for g in range(NGATHER):
            pltpu.make_async_copy(
                table_hbm.at[idx_v.at[slot, g]],
                rows_v.at[slot, pl.ds(g * GLEN, GLEN), :],
                gsems[slot]).wait()

    def compute(slot):
        def pbody(p, carry):
            for v in range(VPR):
                pv = pos_v[p, pl.ds(v * LANES, LANES)]
                for q in range(SEQ_PER_CHUNK):
                    r = q * SEQ + p
                    plsc.addupdate(rows_v.at[slot, r, pl.ds(v * LANES, LANES)],
                                   pv)
            return carry
        lax.fori_loop(0, SEQ, pbody, 0)

    def ostart(j, slot):
        base = (wid * NCHUNK + j) * CHUNK
        pltpu.async_copy(rows_v.at[slot], out_hbm.at[pl.ds(base, CHUNK)],
                         osems[slot])

    def owait(slot):
        pltpu.make_async_copy(rows_v.at[slot], out_hbm.at[pl.ds(0, CHUNK)],
                              osems[slot]).wait()

    fire(0, 0)

    def step(i, carry):
        j0 = 2 * i

        @pl.when(j0 > 0)
        def _():
            owait(1)
        fire(j0 + 1, 1)
        gwait(0)
        compute(0)
        ostart(j0, 0)

        @pl.when(j0 + 2 < NCHUNK)
        def _():
            owait(0)
            fire(j0 + 2, 0)
        gwait(1)
        compute(1)
        ostart(j0 + 1, 1)
        return carry

    lax.fori_loop(0, NCHUNK // 2, step, 0)
    owait(0)
    owait(1)


def kernel(input_ids, item_table, pos_table):
    ids = input_ids.reshape(NW * NCHUNK, NGATHER, GLEN).astype(jnp.int32)
    mesh = plsc.VectorSubcoreMesh(core_axis_name="c", subcore_axis_name="s")
    f = pl.kernel(
        _emb_body,
        out_type=jax.ShapeDtypeStruct((ROWS, HID), jnp.float32),
        mesh=mesh,
        compiler_params=pltpu.CompilerParams(use_tc_tiling_on_sc=False),
        scratch_types=[
            pltpu.VMEM((2, NGATHER, GLEN), jnp.int32),
            pltpu.VMEM((2, CHUNK, HID), jnp.float32),
            pltpu.VMEM((SEQ, HID), jnp.float32),
            pltpu.SemaphoreType.DMA,
            pltpu.SemaphoreType.DMA,
            pltpu.SemaphoreType.DMA,
            pltpu.SemaphoreType.DMA,
        ],
    )
    out = f(ids, item_table, pos_table)
    return out.reshape(BATCH, SEQ, HID)
